# baseline (speedup 1.0000x reference)
import functools

import jax
import jax.numpy as jnp
from jax import lax
from jax.experimental import pallas as pl
from jax.experimental.pallas import tpu as pltpu
from jax.experimental.pallas import tpu_sc as plsc

B, N, C = 4, 20000, 3
PRE, POST, TH = 1024, 512, 0.7
NP = 20480
RR, LL = 160, 128
PK = 32768


def _select_body(cls_ref, score_ref, packed_ref, posm_ref):
    c0 = cls_ref[0]
    c1 = cls_ref[1]
    c2 = cls_ref[2]
    s = jnp.maximum(jnp.maximum(c0, c1), c2)
    lab = jnp.where((c0 >= c1) & (c0 >= c2), 0, jnp.where(c1 >= c2, 1, 2))

    i32 = lax.bitcast_convert_type(s, jnp.int32)
    key = jnp.where(i32 >= 0, i32, i32 ^ jnp.int32(0x7FFFFFFF))
    ukey = lax.bitcast_convert_type(key, jnp.uint32) ^ jnp.uint32(0x80000000)
    rowi = lax.broadcasted_iota(jnp.int32, (RR, LL), 0)
    lanei = lax.broadcasted_iota(jnp.int32, (RR, LL), 1)
    flat = rowi * LL + lanei
    ukey = jnp.where(flat < N, ukey, jnp.uint32(0))

    def bs_body(_, carry):
        lo, hi = carry
        step = jnp.maximum((hi - lo) // jnp.uint32(8), jnp.uint32(1))
        new_lo, new_hi = lo, hi
        for k in range(1, 8):
            m = lo + step * jnp.uint32(k)
            cnt = jnp.sum((ukey >= m).astype(jnp.int32))
            big = cnt >= PRE
            new_lo = jnp.where(big, jnp.maximum(new_lo, m), new_lo)
            new_hi = jnp.where(big, new_hi, jnp.minimum(new_hi, m))
        return new_lo, new_hi

    lo0 = jnp.uint32(0)
    hi0 = jnp.uint32(0xFFFFFFFF)
    t_u, _ = lax.fori_loop(0, 12, bs_body, (lo0, hi0))

    gt = ukey > t_u
    eq = ukey == t_u
    need = PRE - jnp.sum(gt.astype(jnp.int32))

    def ix_body(_, carry):
        lo, hi = carry
        step = jnp.maximum((hi - lo) // 8, 1)
        new_lo, new_hi = lo, hi
        for k in range(1, 8):
            m = lo + step * k
            cnt = jnp.sum((eq & (flat < m)).astype(jnp.int32))
            enough = cnt >= need
            new_hi = jnp.where(enough, jnp.minimum(new_hi, m), new_hi)
            new_lo = jnp.where(enough, new_lo, jnp.maximum(new_lo, m))
        return new_lo, new_hi

    _, m_hi = lax.fori_loop(0, 5, ix_body, (jnp.int32(0), jnp.int32(NP)))
    sel = gt | (eq & (flat < m_hi))
    self32 = sel.astype(jnp.float32)

    tri_r = (
        lax.broadcasted_iota(jnp.int32, (RR, RR), 1)
        < lax.broadcasted_iota(jnp.int32, (RR, RR), 0)
    ).astype(jnp.float32)
    tri_l = (
        lax.broadcasted_iota(jnp.int32, (LL, LL), 0)
        < lax.broadcasted_iota(jnp.int32, (LL, LL), 1)
    ).astype(jnp.float32)
    rowsum = jnp.sum(self32, axis=1, keepdims=True)
    prior = lax.dot_general(
        tri_r, rowsum, (((1,), (0,)), ((), ())),
        preferred_element_type=jnp.float32,
    )
    within = lax.dot_general(
        self32, tri_l, (((1,), (0,)), ((), ())),
        preferred_element_type=jnp.float32,
    )
    pos = (prior + within).astype(jnp.int32)

    score_ref[...] = s
    packed_ref[...] = lab * PK + flat
    posm_ref[...] = jnp.where(sel, pos, PRE)


def _make_compact():
    mesh = plsc.VectorSubcoreMesh(core_axis_name="c", subcore_axis_name="s")

    @functools.partial(
        pl.kernel,
        mesh=mesh,
        compiler_params=pltpu.CompilerParams(needs_layout_passes=False),
        out_type=[
            jax.ShapeDtypeStruct((B, PRE), jnp.int32),
            jax.ShapeDtypeStruct((B, PRE), jnp.float32),
            jax.ShapeDtypeStruct((B, 7 * PRE), jnp.float32),
        ],
        scratch_types=[
            pltpu.VMEM((NP,), jnp.int32),
            pltpu.VMEM((NP,), jnp.int32),
            pltpu.VMEM((NP,), jnp.float32),
            pltpu.VMEM((PRE,), jnp.int32),
            pltpu.VMEM((PRE,), jnp.float32),
            pltpu.VMEM((PRE,), jnp.int32),
            pltpu.VMEM((7 * PRE,), jnp.float32),
            pltpu.SemaphoreType.DMA,
        ],
    )
    def _compact(posm_hbm, pak_hbm, sco_hbm, boxt_hbm, pk_out, sc_out, bx_out,
                 posm_v, pak_v, sco_v, cpk_v, csc_v, gidx_v, bx7_v, sem):
        wid = lax.axis_index("s") * 2 + lax.axis_index("c")

        @pl.when(wid < B)
        def _():
            b = wid
            pltpu.sync_copy(posm_hbm.at[b], posm_v)
            pltpu.sync_copy(pak_hbm.at[b], pak_v)
            pltpu.sync_copy(sco_hbm.at[b], sco_v)

            def scat(i, carry):
                sl = pl.ds(i * 16, 16)
                idxs = posm_v[sl]
                msk = idxs < PRE
                plsc.store_scatter(cpk_v, [idxs], pak_v[sl], mask=msk)
                plsc.store_scatter(csc_v, [idxs], sco_v[sl], mask=msk)
                return carry

            lax.fori_loop(0, NP // 16, scat, 0)

            for c in range(7):
                base = (c * B + b) * N

                def mkidx(i, carry):
                    sl = pl.ds(i * 16, 16)
                    gidx_v[sl] = (cpk_v[sl] % PK) + base
                    return carry

                lax.fori_loop(0, PRE // 16, mkidx, 0)
                pltpu.async_copy(
                    boxt_hbm.at[gidx_v], bx7_v.at[pl.ds(c * PRE, PRE)], sem
                ).wait()

            pltpu.sync_copy(cpk_v, pk_out.at[b])
            pltpu.sync_copy(csc_v, sc_out.at[b])
            pltpu.sync_copy(bx7_v, bx_out.at[b])

    return _compact


_compact_kernel = _make_compact()


def _nms_body(bx_ref, cpk_ref, csc_ref, out_ref):
    sc_row = csc_ref[...]
    cpk = cpk_ref[...]
    ix_row = (cpk % PK).astype(jnp.float32)
    lb_row = (cpk // PK).astype(jnp.float32)
    pt = jnp.concatenate(
        [bx_ref[...], sc_row, lb_row, ix_row, jnp.zeros((6, PRE), jnp.float32)],
        axis=0,
    )
    payload = jnp.transpose(pt)
    sc_col = payload[:, 7:8]
    ix_col = payload[:, 9:10]

    cmp = jnp.where(
        (sc_col > sc_row) | ((sc_col == sc_row) & (ix_col < ix_row)), 1.0, 0.0
    )

    x = pt[0:1, :]
    y = pt[1:2, :]
    dx = pt[3:4, :]
    dy = pt[4:5, :]
    ry = pt[6:7, :]
    c = jnp.abs(jnp.cos(ry))
    s = jnp.abs(jnp.sin(ry))
    hx = (dx * c + dy * s) * 0.5
    hy = (dx * s + dy * c) * 0.5
    a_row = jnp.concatenate(
        [x - hx, y - hy, x + hx, y + hy, jnp.zeros((4, PRE), jnp.float32)], axis=0
    )
    a_col = jnp.transpose(a_row)

    x1 = jnp.maximum(a_col[:, 0:1], a_row[0:1, :])
    y1 = jnp.maximum(a_col[:, 1:2], a_row[1:2, :])
    x2 = jnp.minimum(a_col[:, 2:3], a_row[2:3, :])
    y2 = jnp.minimum(a_col[:, 3:4], a_row[3:4, :])
    inter = jnp.clip(x2 - x1, 0.0, None) * jnp.clip(y2 - y1, 0.0, None)
    area_row = (a_row[2:3, :] - a_row[0:1, :]) * (a_row[3:4, :] - a_row[1:2, :])
    area_col = (a_col[:, 2:3] - a_col[:, 0:1]) * (a_col[:, 3:4] - a_col[:, 1:2])
    union = area_col + area_row - inter
    iou = inter / (union + 1e-6)

    sup = jnp.where(iou > TH, cmp, 0.0)

    def cond(carry):
        _, changed = carry
        return changed

    def body(carry):
        keep, _ = carry
        v = lax.dot_general(
            keep, sup, (((1,), (0,)), ((), ())),
            preferred_element_type=jnp.float32,
        )
        new = (v < 0.5).astype(jnp.float32)
        changed = jnp.sum(jnp.abs(new - keep)) > 0.0
        return new, changed

    keep0 = jnp.ones((1, PRE), jnp.float32)
    keep, _ = lax.while_loop(cond, body, (keep0, jnp.bool_(True)))

    pos = lax.dot_general(
        keep, cmp, (((1,), (0,)), ((), ())),
        preferred_element_type=jnp.float32,
    )
    slot = lax.broadcasted_iota(jnp.int32, (POST, PRE), 0).astype(jnp.float32)
    selm = jnp.where((jnp.abs(pos - slot) < 0.5) & (keep > 0.5), 1.0, 0.0)
    out_ref[...] = lax.dot_general(
        selm, payload, (((1,), (0,)), ((), ())),
        preferred_element_type=jnp.float32, precision=lax.Precision.HIGHEST,
    )


@jax.jit
def kernel(batch_box_preds, batch_cls_preds):
    cls_t = jnp.swapaxes(batch_cls_preds, 1, 2)
    cls_t = jnp.concatenate(
        [cls_t, jnp.zeros((B, C, NP - N), jnp.float32)], axis=-1
    ).reshape(B, C, RR, LL)

    score_g, packed_g, posm_g = pl.pallas_call(
        _select_body,
        grid=(B,),
        in_specs=[pl.BlockSpec((None, C, RR, LL), lambda b: (b, 0, 0, 0))],
        out_specs=[
            pl.BlockSpec((None, RR, LL), lambda b: (b, 0, 0)),
            pl.BlockSpec((None, RR, LL), lambda b: (b, 0, 0)),
            pl.BlockSpec((None, RR, LL), lambda b: (b, 0, 0)),
        ],
        out_shape=[
            jax.ShapeDtypeStruct((B, RR, LL), jnp.float32),
            jax.ShapeDtypeStruct((B, RR, LL), jnp.int32),
            jax.ShapeDtypeStruct((B, RR, LL), jnp.int32),
        ],
    )(cls_t)

    boxt = jnp.transpose(batch_box_preds, (2, 0, 1)).reshape(7 * B * N)
    cpk, csc, bxg = _compact_kernel(
        posm_g.reshape(B, NP), packed_g.reshape(B, NP), score_g.reshape(B, NP),
        boxt,
    )
    bxg = bxg.reshape(B, 7, PRE)

    out = pl.pallas_call(
        _nms_body,
        grid=(B,),
        in_specs=[
            pl.BlockSpec((None, 7, PRE), lambda b: (b, 0, 0)),
            pl.BlockSpec((None, 1, PRE), lambda b: (b, 0, 0)),
            pl.BlockSpec((None, 1, PRE), lambda b: (b, 0, 0)),
        ],
        out_specs=pl.BlockSpec((None, POST, 16), lambda b: (b, 0, 0)),
        out_shape=jax.ShapeDtypeStruct((B, POST, 16), jnp.float32),
    )(bxg, cpk.reshape(B, 1, PRE), csc.reshape(B, 1, PRE))

    rois = out[..., :7]
    roi_scores = out[..., 7]
    roi_labels = jnp.round(out[..., 8]).astype(jnp.int32) + 1
    return rois, roi_scores, roi_labels


# speedup vs baseline: 1.0643x; 1.0643x over previous
import functools

import jax
import jax.numpy as jnp
from jax import lax
from jax.experimental import pallas as pl
from jax.experimental.pallas import tpu as pltpu
from jax.experimental.pallas import tpu_sc as plsc

B, N, C = 4, 20000, 3
PRE, POST, TH = 1024, 512, 0.7
NP = 20480
RR, LL = 160, 128
PK = 32768


def _select_body(cls_ref, score_ref, packed_ref, posm_ref):
    c0 = cls_ref[0]
    c1 = cls_ref[1]
    c2 = cls_ref[2]
    s = jnp.maximum(jnp.maximum(c0, c1), c2)
    lab = jnp.where((c0 >= c1) & (c0 >= c2), 0, jnp.where(c1 >= c2, 1, 2))

    i32 = lax.bitcast_convert_type(s, jnp.int32)
    key = jnp.where(i32 >= 0, i32, i32 ^ jnp.int32(0x7FFFFFFF))
    ukey = lax.bitcast_convert_type(key, jnp.uint32) ^ jnp.uint32(0x80000000)
    rowi = lax.broadcasted_iota(jnp.int32, (RR, LL), 0)
    lanei = lax.broadcasted_iota(jnp.int32, (RR, LL), 1)
    flat = rowi * LL + lanei
    ukey = jnp.where(flat < N, ukey, jnp.uint32(0))

    def bs_body(_, carry):
        lo, hi = carry
        step = jnp.maximum((hi - lo) // jnp.uint32(8), jnp.uint32(1))
        new_lo, new_hi = lo, hi
        for k in range(1, 8):
            m = lo + step * jnp.uint32(k)
            cnt = jnp.sum((ukey >= m).astype(jnp.int32))
            big = cnt >= PRE
            new_lo = jnp.where(big, jnp.maximum(new_lo, m), new_lo)
            new_hi = jnp.where(big, new_hi, jnp.minimum(new_hi, m))
        return new_lo, new_hi

    lo0 = jnp.uint32(0)
    hi0 = jnp.uint32(0xFFFFFFFF)
    t_u, _ = lax.fori_loop(0, 12, bs_body, (lo0, hi0))

    gt = ukey > t_u
    eq = ukey == t_u
    need = PRE - jnp.sum(gt.astype(jnp.int32))

    def ix_body(_, carry):
        lo, hi = carry
        step = jnp.maximum((hi - lo) // 8, 1)
        new_lo, new_hi = lo, hi
        for k in range(1, 8):
            m = lo + step * k
            cnt = jnp.sum((eq & (flat < m)).astype(jnp.int32))
            enough = cnt >= need
            new_hi = jnp.where(enough, jnp.minimum(new_hi, m), new_hi)
            new_lo = jnp.where(enough, new_lo, jnp.maximum(new_lo, m))
        return new_lo, new_hi

    _, m_hi = lax.fori_loop(0, 5, ix_body, (jnp.int32(0), jnp.int32(NP)))
    sel = gt | (eq & (flat < m_hi))
    self32 = sel.astype(jnp.float32)

    tri_r = (
        lax.broadcasted_iota(jnp.int32, (RR, RR), 1)
        < lax.broadcasted_iota(jnp.int32, (RR, RR), 0)
    ).astype(jnp.float32)
    tri_l = (
        lax.broadcasted_iota(jnp.int32, (LL, LL), 0)
        < lax.broadcasted_iota(jnp.int32, (LL, LL), 1)
    ).astype(jnp.float32)
    rowsum = jnp.sum(self32, axis=1, keepdims=True)
    prior = lax.dot_general(
        tri_r, rowsum, (((1,), (0,)), ((), ())),
        preferred_element_type=jnp.float32,
    )
    within = lax.dot_general(
        self32, tri_l, (((1,), (0,)), ((), ())),
        preferred_element_type=jnp.float32,
    )
    pos = (prior + within).astype(jnp.int32)

    score_ref[...] = s
    packed_ref[...] = lab * PK + flat
    posm_ref[...] = jnp.where(sel, pos, PRE)


def _make_compact():
    mesh = plsc.VectorSubcoreMesh(core_axis_name="c", subcore_axis_name="s")

    @functools.partial(
        pl.kernel,
        mesh=mesh,
        compiler_params=pltpu.CompilerParams(needs_layout_passes=False),
        out_type=[
            jax.ShapeDtypeStruct((B, PRE), jnp.int32),
            jax.ShapeDtypeStruct((B, PRE), jnp.float32),
        ],
        scratch_types=[
            pltpu.VMEM((NP,), jnp.int32),
            pltpu.VMEM((NP,), jnp.int32),
            pltpu.VMEM((NP,), jnp.float32),
            pltpu.VMEM((PRE,), jnp.int32),
            pltpu.VMEM((PRE,), jnp.float32),
        ],
    )
    def _compact(posm_hbm, pak_hbm, sco_hbm, pk_out, sc_out,
                 posm_v, pak_v, sco_v, cpk_v, csc_v):
        wid = lax.axis_index("s") * 2 + lax.axis_index("c")

        @pl.when(wid < B)
        def _():
            b = wid
            pltpu.sync_copy(posm_hbm.at[b], posm_v)
            pltpu.sync_copy(pak_hbm.at[b], pak_v)
            pltpu.sync_copy(sco_hbm.at[b], sco_v)

            def scat(i, carry):
                sl = pl.ds(i * 16, 16)
                idxs = posm_v[sl]
                msk = idxs < PRE
                plsc.store_scatter(cpk_v, [idxs], pak_v[sl], mask=msk)
                plsc.store_scatter(csc_v, [idxs], sco_v[sl], mask=msk)
                return carry

            lax.fori_loop(0, NP // 16, scat, 0)
            pltpu.sync_copy(cpk_v, pk_out.at[b])
            pltpu.sync_copy(csc_v, sc_out.at[b])

    return _compact


_compact_kernel = _make_compact()


def _nms_body(bx_ref, cpk_ref, csc_ref, out_ref):
    sc_row = csc_ref[...]
    cpk = cpk_ref[...]
    ix_row = (cpk % PK).astype(jnp.float32)
    lb_row = (cpk // PK).astype(jnp.float32)
    bxt = jnp.transpose(bx_ref[...])
    pt = jnp.concatenate(
        [bxt, sc_row, lb_row, ix_row, jnp.zeros((6, PRE), jnp.float32)], axis=0
    )
    payload = jnp.transpose(pt)
    sc_col = payload[:, 7:8]
    ix_col = payload[:, 9:10]

    cmp = jnp.where(
        (sc_col > sc_row) | ((sc_col == sc_row) & (ix_col < ix_row)), 1.0, 0.0
    )

    x = pt[0:1, :]
    y = pt[1:2, :]
    dx = pt[3:4, :]
    dy = pt[4:5, :]
    ry = pt[6:7, :]
    c = jnp.abs(jnp.cos(ry))
    s = jnp.abs(jnp.sin(ry))
    hx = (dx * c + dy * s) * 0.5
    hy = (dx * s + dy * c) * 0.5
    a_row = jnp.concatenate(
        [x - hx, y - hy, x + hx, y + hy, jnp.zeros((4, PRE), jnp.float32)], axis=0
    )
    a_col = jnp.transpose(a_row)

    x1 = jnp.maximum(a_col[:, 0:1], a_row[0:1, :])
    y1 = jnp.maximum(a_col[:, 1:2], a_row[1:2, :])
    x2 = jnp.minimum(a_col[:, 2:3], a_row[2:3, :])
    y2 = jnp.minimum(a_col[:, 3:4], a_row[3:4, :])
    inter = jnp.clip(x2 - x1, 0.0, None) * jnp.clip(y2 - y1, 0.0, None)
    area_row = (a_row[2:3, :] - a_row[0:1, :]) * (a_row[3:4, :] - a_row[1:2, :])
    area_col = (a_col[:, 2:3] - a_col[:, 0:1]) * (a_col[:, 3:4] - a_col[:, 1:2])
    union = area_col + area_row - inter
    iou = inter / (union + 1e-6)

    sup = jnp.where(iou > TH, cmp, 0.0)

    def cond(carry):
        _, changed = carry
        return changed

    def body(carry):
        keep, _ = carry
        v = lax.dot_general(
            keep, sup, (((1,), (0,)), ((), ())),
            preferred_element_type=jnp.float32,
        )
        new = (v < 0.5).astype(jnp.float32)
        changed = jnp.sum(jnp.abs(new - keep)) > 0.0
        return new, changed

    keep0 = jnp.ones((1, PRE), jnp.float32)
    keep, _ = lax.while_loop(cond, body, (keep0, jnp.bool_(True)))

    pos = lax.dot_general(
        keep, cmp, (((1,), (0,)), ((), ())),
        preferred_element_type=jnp.float32,
    )
    slot = lax.broadcasted_iota(jnp.int32, (POST, PRE), 0).astype(jnp.float32)
    selm = jnp.where((jnp.abs(pos - slot) < 0.5) & (keep > 0.5), 1.0, 0.0)
    out_ref[...] = lax.dot_general(
        selm, payload, (((1,), (0,)), ((), ())),
        preferred_element_type=jnp.float32, precision=lax.Precision.HIGHEST,
    )


@jax.jit
def kernel(batch_box_preds, batch_cls_preds):
    cls_t = jnp.swapaxes(batch_cls_preds, 1, 2)
    cls_t = jnp.concatenate(
        [cls_t, jnp.zeros((B, C, NP - N), jnp.float32)], axis=-1
    ).reshape(B, C, RR, LL)

    score_g, packed_g, posm_g = pl.pallas_call(
        _select_body,
        grid=(B,),
        in_specs=[pl.BlockSpec((None, C, RR, LL), lambda b: (b, 0, 0, 0))],
        out_specs=[
            pl.BlockSpec((None, RR, LL), lambda b: (b, 0, 0)),
            pl.BlockSpec((None, RR, LL), lambda b: (b, 0, 0)),
            pl.BlockSpec((None, RR, LL), lambda b: (b, 0, 0)),
        ],
        out_shape=[
            jax.ShapeDtypeStruct((B, RR, LL), jnp.float32),
            jax.ShapeDtypeStruct((B, RR, LL), jnp.int32),
            jax.ShapeDtypeStruct((B, RR, LL), jnp.int32),
        ],
    )(cls_t)

    cpk, csc = _compact_kernel(
        posm_g.reshape(B, NP), packed_g.reshape(B, NP), score_g.reshape(B, NP)
    )
    cidx = cpk % PK
    bxg = jnp.take_along_axis(batch_box_preds, cidx[..., None], axis=1)

    out = pl.pallas_call(
        _nms_body,
        grid=(B,),
        in_specs=[
            pl.BlockSpec((None, PRE, 7), lambda b: (b, 0, 0)),
            pl.BlockSpec((None, 1, PRE), lambda b: (b, 0, 0)),
            pl.BlockSpec((None, 1, PRE), lambda b: (b, 0, 0)),
        ],
        out_specs=pl.BlockSpec((None, POST, 16), lambda b: (b, 0, 0)),
        out_shape=jax.ShapeDtypeStruct((B, POST, 16), jnp.float32),
    )(bxg, cpk.reshape(B, 1, PRE), csc.reshape(B, 1, PRE))

    rois = out[..., :7]
    roi_scores = out[..., 7]
    roi_labels = jnp.round(out[..., 8]).astype(jnp.int32) + 1
    return rois, roi_scores, roi_labels
